# baseline (speedup 1.0000x reference)
import functools

import jax
import jax.numpy as jnp
from jax import lax
from jax.experimental import pallas as pl
from jax.experimental.pallas import tpu as pltpu
from jax.experimental.pallas import tpu_sc as plsc

_B = 16384
_D = 32
_L = 16
_NC = 2
_NS = 16
_NW = _NC * _NS
_BPW = _B // _NW
_CH = 128
_NCH = _BPW // _CH


def _build():
    mesh = plsc.VectorSubcoreMesh(core_axis_name="c", subcore_axis_name="s")

    @functools.partial(
        pl.kernel,
        mesh=mesh,
        out_type=jax.ShapeDtypeStruct((_B,), jnp.float32),
        scratch_types=[
            pltpu.VMEM((_NCH, _CH), jnp.int32),
            pltpu.VMEM((_NCH, _CH), jnp.int32),
            pltpu.VMEM((_BPW, _D), jnp.float32),
            pltpu.VMEM((_BPW, _D), jnp.float32),
            pltpu.VMEM((_BPW,), jnp.float32),
            pltpu.SemaphoreType.DMA,
        ],
        compiler_params=pltpu.CompilerParams(
            needs_layout_passes=False, use_tc_tiling_on_sc=False),
    )
    def k(uf_hbm, if_hbm, uidx_hbm, iidx_hbm, out_hbm,
          uidx_v, iidx_v, urows_v, irows_v, out_v, sem):
        wid = lax.axis_index("s") * _NC + lax.axis_index("c")

        pltpu.sync_copy(uidx_hbm.at[pl.ds(wid * _NCH, _NCH)], uidx_v)
        pltpu.sync_copy(iidx_hbm.at[pl.ds(wid * _NCH, _NCH)], iidx_v)

        copies = []
        for j in range(_NCH):
            copies.append(pltpu.async_copy(
                uf_hbm.at[uidx_v.at[j]],
                urows_v.at[pl.ds(j * _CH, _CH)], sem))
            copies.append(pltpu.async_copy(
                if_hbm.at[iidx_v.at[j]],
                irows_v.at[pl.ds(j * _CH, _CH)], sem))
        for c in copies:
            c.wait()

        lane = lax.iota(jnp.int32, _L)

        def body(b, carry):
            row = lane + b * _L
            acc = jnp.zeros((_L,), jnp.float32)
            for d in range(_D):
                col = jnp.full((_L,), d, jnp.int32)
                acc = acc + plsc.load_gather(urows_v, [row, col]) * \
                    plsc.load_gather(irows_v, [row, col])
            out_v[pl.ds(b * _L, _L)] = 1.0 / (1.0 + jnp.exp(-acc))
            return carry

        lax.fori_loop(0, _BPW // _L, body, 0)

        pltpu.sync_copy(out_v, out_hbm.at[pl.ds(wid * _BPW, _BPW)])

    return k


_kernel_call = _build()


def kernel(X, user_factors, item_factors):
    Xi = X.astype(jnp.int32)
    uidx = Xi[:, 0].reshape(_NW * _NCH, _CH)
    iidx = Xi[:, 1].reshape(_NW * _NCH, _CH)
    out = _kernel_call(user_factors, item_factors, uidx, iidx)
    return out.reshape(_B, 1)
